# baseline (speedup 1.0000x reference)
import functools
import math

import jax
import jax.numpy as jnp
from jax import lax
from jax.experimental import pallas as pl
from jax.experimental.pallas import tpu as pltpu
from jax.experimental.pallas import tpu_sc as plsc

_M = 4
_COS_M = math.cos(_M)
_SIN_M = math.sin(_M)

_LN = 16
_CH = 10000
_UNR = 5


def _sc_reduce(x, lbl_i32):
    b, c = x.shape
    info = plsc.get_sparse_core_info()
    nw = info.num_cores * info.num_subcores
    rpw = b // nw
    cpr = c // _CH
    nt = rpw * cpr
    nv = _CH // _LN
    ni = nv // _UNR
    assert c % _CH == 0 and nv % _UNR == 0 and nt % 2 == 0

    mesh = plsc.VectorSubcoreMesh(core_axis_name="c", subcore_axis_name="s")

    @functools.partial(
        pl.kernel,
        mesh=mesh,
        out_type=(
            jax.ShapeDtypeStruct((nw, _LN), jnp.float32),
            jax.ShapeDtypeStruct((b, _LN), jnp.float32),
        ),
        scratch_types=[
            pltpu.VMEM((_CH,), jnp.float32),
            pltpu.VMEM((_CH,), jnp.float32),
            pltpu.VMEM((rpw,), jnp.int32),
            pltpu.VMEM((rpw, _LN), jnp.float32),
            pltpu.VMEM((_LN,), jnp.float32),
            pltpu.SemaphoreType.DMA,
            pltpu.SemaphoreType.DMA,
            pltpu.SemaphoreType.DMA,
        ],
        compiler_params=pltpu.CompilerParams(use_tc_tiling_on_sc=False, needs_layout_passes=False),
    )
    def k(x_hbm, lbl_hbm, part_hbm, rows_hbm,
          buf0, buf1, lblv, gbuf, partv, sem0, sem1, semg):
        wid = lax.axis_index("s") * info.num_cores + lax.axis_index("c")
        r0 = wid * rpw

        def start(u, buf, sem):
            row = r0 + u // cpr
            cc = (u % cpr) * _CH
            pltpu.async_copy(x_hbm.at[row].at[pl.ds(cc, _CH)], buf, sem)

        def wait(buf, sem):
            pltpu.make_async_copy(x_hbm.at[r0].at[pl.ds(0, _CH)], buf,
                                  sem).wait()

        def compute(buf, acc):
            def inner(i, accs):
                base = i * (_LN * _UNR)
                out = []
                for u in range(_UNR):
                    v = buf[pl.ds(base + u * _LN, _LN)]
                    out.append(accs[u] + v * v)
                return tuple(out)
            accs = lax.fori_loop(
                0, ni, inner,
                tuple(jnp.zeros((_LN,), jnp.float32) for _ in range(_UNR)))
            for a in accs:
                acc = acc + a
            return acc

        start(0, buf0, sem0)
        start(1, buf1, sem1)

        def outer(i, acc):
            t0 = i * 2
            wait(buf0, sem0)
            acc = compute(buf0, acc)

            @pl.when(t0 + 2 < nt)
            def _():
                start(t0 + 2, buf0, sem0)

            wait(buf1, sem1)
            acc = compute(buf1, acc)

            @pl.when(t0 + 3 < nt)
            def _():
                start(t0 + 3, buf1, sem1)

            return acc

        acc = lax.fori_loop(0, nt // 2, outer, jnp.zeros((_LN,), jnp.float32))

        partv[...] = acc
        pltpu.sync_copy(partv, part_hbm.at[wid])
        pltpu.sync_copy(gbuf, rows_hbm.at[pl.ds(r0, rpw)])

    return k(x, lbl_i32)


def _tc_fin(part_ref, rows_ref, lbl_ref, out_ref, *, n):
    total = jnp.sum(part_ref[...])
    rows = rows_ref[...]
    lane = jax.lax.broadcasted_iota(jnp.int32, rows.shape, 1)
    off = lbl_ref[...] % _LN
    v = jnp.sum(jnp.where(lane == off, rows, 0.0), axis=1, keepdims=True)
    phi = v * _COS_M - jnp.sqrt(jnp.maximum(1.0 - v * v, 0.0)) * _SIN_M
    corr = jnp.sum(phi * phi - v * v)
    out_ref[0, 0, 0] = (total + corr) * (_M * _M / n)


def kernel(input, label):
    b, c = input.shape
    lbl = label.astype(jnp.int32)
    part, rows = _sc_reduce(input, lbl)

    out = pl.pallas_call(
        functools.partial(_tc_fin, n=b * c),
        out_specs=pl.BlockSpec((1, 1, 1), lambda: (0, 0, 0),
                               memory_space=pltpu.SMEM),
        out_shape=jax.ShapeDtypeStruct((1, 1, 1), jnp.float32),
    )(part, rows, lbl.reshape(b, 1))
    return out.reshape(())


# speedup vs baseline: 1.9146x; 1.9146x over previous
import functools
import math

import jax
import jax.numpy as jnp
from jax import lax
from jax.experimental import pallas as pl
from jax.experimental.pallas import tpu as pltpu
from jax.experimental.pallas import tpu_sc as plsc

_M = 4
_COS_M = math.cos(_M)
_SIN_M = math.sin(_M)

_LN = 16
_TILE = 128
_W = 5248
_TPB = 19
_UNR = 8


def _sc_reduce(x, lbl_i32, tail0):
    b, c = x.shape
    info = plsc.get_sparse_core_info()
    nw = info.num_cores * info.num_subcores
    rpw = b // nw
    nbd = rpw // 8
    nt = nbd * _TPB
    nvr = _W // _LN
    ni = nvr // _UNR
    assert tail0 == _TPB * _W and nvr % _UNR == 0 and nt % 2 == 0
    assert b % (nw * 8) == 0 and b % _TILE == 0

    mesh = plsc.VectorSubcoreMesh(core_axis_name="c", subcore_axis_name="s")

    @functools.partial(
        pl.kernel,
        mesh=mesh,
        out_type=(
            jax.ShapeDtypeStruct((nw, 8, _TILE), jnp.float32),
            jax.ShapeDtypeStruct((b, 8, _TILE), jnp.float32),
        ),
        scratch_types=[
            pltpu.VMEM((8, _W), jnp.float32),
            pltpu.VMEM((8, _W), jnp.float32),
            pltpu.VMEM((_TILE,), jnp.int32),
            pltpu.VMEM((8, _TILE), jnp.float32),
            pltpu.SemaphoreType.DMA,
            pltpu.SemaphoreType.DMA,
            pltpu.SemaphoreType.DMA,
        ],
        compiler_params=pltpu.CompilerParams(needs_layout_passes=False),
    )
    def k(x_hbm, lbl_hbm, part_hbm, tiles_hbm,
          buf0, buf1, lblv, partv, sem0, sem1, semg):
        wid = lax.axis_index("s") * info.num_cores + lax.axis_index("c")
        r0 = wid * rpw

        pltpu.sync_copy(lbl_hbm.at[pl.ds((wid // 4) * _TILE, _TILE)], lblv)
        loff = (wid % 4) * rpw
        lane_iota = lax.broadcasted_iota(jnp.int32, (_LN,), 0)
        clamp = ((c - _TILE) // _TILE) * _TILE
        gds = []
        for t in range(rpw):
            vec = lblv[pl.ds(loff + (t // _LN) * _LN, _LN)]
            sel = jnp.where(lane_iota == (t % _LN), vec, 0)
            s = jnp.max(sel)
            col0 = jnp.minimum((s // _TILE) * _TILE, clamp)
            band_r = r0 + (t // 8) * 8
            gds.append(pltpu.async_copy(
                x_hbm.at[pl.ds(band_r, 8), pl.ds(col0, _TILE)],
                tiles_hbm.at[r0 + t], semg))

        def start(u, buf, sem):
            band = u // _TPB
            j = u % _TPB
            pltpu.async_copy(
                x_hbm.at[pl.ds(r0 + band * 8, 8), pl.ds(j * _W, _W)],
                buf, sem)

        def wait(buf, sem):
            pltpu.make_async_copy(
                x_hbm.at[pl.ds(r0, 8), pl.ds(0, _W)], buf, sem).wait()

        def compute(buf, acc):
            def row_loop(r):
                def inner(i, accs):
                    base = i * (_LN * _UNR)
                    out = []
                    for u in range(_UNR):
                        v = buf[r, pl.ds(base + u * _LN, _LN)]
                        out.append(accs[u] + v * v)
                    return tuple(out)
                return inner
            accs = tuple(jnp.zeros((_LN,), jnp.float32) for _ in range(_UNR))
            for r in range(8):
                accs = lax.fori_loop(0, ni, row_loop(r), accs)
            for a in accs:
                acc = acc + a
            return acc

        start(0, buf0, sem0)
        start(1, buf1, sem1)

        def outer(i, acc):
            t0 = i * 2
            wait(buf0, sem0)
            acc = compute(buf0, acc)

            @pl.when(t0 + 2 < nt)
            def _():
                start(t0 + 2, buf0, sem0)

            wait(buf1, sem1)
            acc = compute(buf1, acc)

            @pl.when(t0 + 3 < nt)
            def _():
                start(t0 + 3, buf1, sem1)

            return acc

        acc = lax.fori_loop(0, nt // 2, outer, jnp.zeros((_LN,), jnp.float32))

        for d in gds:
            d.wait()
        partv[0, pl.ds(0, _LN)] = acc
        pltpu.sync_copy(partv, part_hbm.at[wid])

    return k(x, lbl_i32)


def _tc_fin(part_ref, tiles_ref, tail_ref, lbl_ref, out_ref, *, c, tail0, n):
    part = part_ref[...]
    p1 = jax.lax.broadcasted_iota(jnp.int32, part.shape, 1)
    p2 = jax.lax.broadcasted_iota(jnp.int32, part.shape, 2)
    total = jnp.sum(jnp.where((p1 == 0) & (p2 < _LN), part, 0.0))

    tail = tail_ref[...]
    total += jnp.sum(tail * tail)

    lbl = lbl_ref[...]
    ct = jax.lax.broadcasted_iota(jnp.int32, tail.shape, 1)
    v_tail = jnp.sum(jnp.where(ct == lbl - tail0, tail, 0.0), axis=1,
                     keepdims=True)

    tiles = tiles_ref[...]
    clamp = ((c - _TILE) // _TILE) * _TILE
    off = lbl - jnp.minimum((lbl // _TILE) * _TILE, clamp)
    t0i = jax.lax.broadcasted_iota(jnp.int32, tiles.shape, 0)
    t1i = jax.lax.broadcasted_iota(jnp.int32, tiles.shape, 1)
    t2i = jax.lax.broadcasted_iota(jnp.int32, tiles.shape, 2)
    m = ((t1i == t0i % 8) & (t2i == off.reshape(-1, 1, 1))
         & (lbl.reshape(-1, 1, 1) < tail0))
    v_sc = jnp.sum(jnp.where(m, tiles, 0.0), axis=(1, 2)).reshape(-1, 1)

    v = v_tail + v_sc
    phi = v * _COS_M - jnp.sqrt(jnp.maximum(1.0 - v * v, 0.0)) * _SIN_M
    corr = jnp.sum(phi * phi - v * v)
    out_ref[0, 0, 0] = (total + corr) * (_M * _M / n)


def kernel(input, label):
    b, c = input.shape
    tail0 = _TPB * _W
    lbl = label.astype(jnp.int32)
    part, tiles = _sc_reduce(input, lbl, tail0)
    tail = jax.lax.slice(input, (0, tail0), (b, c))

    out = pl.pallas_call(
        functools.partial(_tc_fin, c=c, tail0=tail0, n=b * c),
        out_specs=pl.BlockSpec((1, 1, 1), lambda: (0, 0, 0),
                               memory_space=pltpu.SMEM),
        out_shape=jax.ShapeDtypeStruct((1, 1, 1), jnp.float32),
    )(part, tiles, tail, lbl.reshape(b, 1))
    return out.reshape(())


# speedup vs baseline: 6.8896x; 3.5985x over previous
import functools
import math

import jax
import jax.numpy as jnp
from jax.experimental import pallas as pl
from jax.experimental.pallas import tpu as pltpu

_M = 4
_COS_M = math.cos(_M)
_SIN_M = math.sin(_M)


def _tc_body(x_ref, lbl_ref, out_ref, acc_ref, gacc_ref, *, c):
    j = pl.program_id(0)
    nj = pl.num_programs(0)
    br = x_ref.shape[0]

    @pl.when(j == 0)
    def _init():
        acc_ref[0, 0] = 0.0
        gacc_ref[...] = jnp.zeros_like(gacc_ref)

    x = x_ref[...]
    acc_ref[0, 0] += jnp.sum(x * x)

    row = jax.lax.broadcasted_iota(jnp.int32, x.shape, 0)
    m = row == lbl_ref[...] - j * br
    gacc_ref[...] += jnp.sum(jnp.where(m, x, 0.0), axis=0, keepdims=True)

    @pl.when(j == nj - 1)
    def _fin():
        v = gacc_ref[...]
        phi = v * _COS_M - jnp.sqrt(jnp.maximum(1.0 - v * v, 0.0)) * _SIN_M
        corr = jnp.sum(phi * phi - v * v)
        total_n = gacc_ref.shape[1] * c
        out_ref[0, 0, 0] = (acc_ref[0, 0] + corr) * (_M * _M / total_n)


def kernel(input, label):
    b, c = input.shape
    xt = input.T
    br = 5000
    assert c % br == 0 and br % 8 == 0
    grid = (c // br,)
    lbl = label.astype(jnp.int32).reshape(1, b)

    out = pl.pallas_call(
        functools.partial(_tc_body, c=c),
        grid=grid,
        in_specs=[
            pl.BlockSpec((br, b), lambda j: (j, 0)),
            pl.BlockSpec((1, b), lambda j: (0, 0)),
        ],
        out_specs=pl.BlockSpec((1, 1, 1), lambda j: (0, 0, 0),
                               memory_space=pltpu.SMEM),
        out_shape=jax.ShapeDtypeStruct((1, 1, 1), jnp.float32),
        scratch_shapes=[
            pltpu.SMEM((1, 1), jnp.float32),
            pltpu.VMEM((1, b), jnp.float32),
        ],
    )(xt, lbl)
    return out.reshape(())


# speedup vs baseline: 7.0407x; 1.0219x over previous
import functools
import math

import jax
import jax.numpy as jnp
from jax import lax
from jax.experimental import pallas as pl
from jax.experimental.pallas import tpu as pltpu
from jax.experimental.pallas import tpu_sc as plsc

_M = 4
_COS_M = math.cos(_M)
_SIN_M = math.sin(_M)

_LN = 16
_TILE = 128


def _sc_gather_tiles(xt, lbl_i32):
    c, b = xt.shape
    info = plsc.get_sparse_core_info()
    nw = info.num_cores * info.num_subcores
    rpw = b // nw
    assert rpw * (nw // 4) * 4 == b and (rpw * 4) % _TILE == 0

    mesh = plsc.VectorSubcoreMesh(core_axis_name="c", subcore_axis_name="s")

    @functools.partial(
        pl.kernel,
        mesh=mesh,
        out_type=jax.ShapeDtypeStruct((b, 8, _TILE), jnp.float32),
        scratch_types=[
            pltpu.VMEM((_TILE,), jnp.int32),
            pltpu.SemaphoreType.DMA,
        ],
        compiler_params=pltpu.CompilerParams(needs_layout_passes=False),
    )
    def k(xt_hbm, lbl_hbm, tiles_hbm, lblv, semg):
        wid = lax.axis_index("s") * info.num_cores + lax.axis_index("c")
        i0 = wid * rpw
        col0 = (wid // 4) * _TILE
        pltpu.sync_copy(lbl_hbm.at[pl.ds((wid // 4) * _TILE, _TILE)], lblv)
        loff = (wid % 4) * rpw
        lane_iota = lax.broadcasted_iota(jnp.int32, (_LN,), 0)
        gds = []
        for t in range(rpw):
            vec = lblv[pl.ds(loff + (t // _LN) * _LN, _LN)]
            sel = jnp.where(lane_iota == (t % _LN), vec, 0)
            s = jnp.max(sel)
            gds.append(pltpu.async_copy(
                xt_hbm.at[pl.ds((s // 8) * 8, 8), pl.ds(col0, _TILE)],
                tiles_hbm.at[i0 + t], semg))
        for d in gds:
            d.wait()

    return k(xt, lbl_i32)


def _tc_sum(x_ref, out_ref, acc_ref):
    j = pl.program_id(0)
    nj = pl.num_programs(0)

    @pl.when(j == 0)
    def _init():
        acc_ref[0, 0] = 0.0

    x = x_ref[...]
    acc_ref[0, 0] += jnp.sum(x * x)

    @pl.when(j == nj - 1)
    def _out():
        out_ref[0, 0, 0] = acc_ref[0, 0]


def _tc_fin(part_ref, tiles_ref, lbl_ref, out_ref, *, n):
    total = part_ref[0, 0, 0]
    tiles = tiles_ref[...]
    lbl = lbl_ref[...]
    t0i = jax.lax.broadcasted_iota(jnp.int32, tiles.shape, 0)
    t1i = jax.lax.broadcasted_iota(jnp.int32, tiles.shape, 1)
    t2i = jax.lax.broadcasted_iota(jnp.int32, tiles.shape, 2)
    m = (t1i == lbl.reshape(-1, 1, 1) % 8) & (t2i == t0i % _TILE)
    v = jnp.sum(jnp.where(m, tiles, 0.0), axis=(1, 2)).reshape(-1, 1)
    phi = v * _COS_M - jnp.sqrt(jnp.maximum(1.0 - v * v, 0.0)) * _SIN_M
    corr = jnp.sum(phi * phi - v * v)
    out_ref[0, 0, 0] = (total + corr) * (_M * _M / n)


def kernel(input, label):
    b, c = input.shape
    xt = input.T
    lbl = label.astype(jnp.int32)

    tiles = _sc_gather_tiles(xt, lbl)

    br = 5000
    assert c % br == 0 and br % 8 == 0
    part = pl.pallas_call(
        _tc_sum,
        grid=(c // br,),
        in_specs=[pl.BlockSpec((br, b), lambda j: (j, 0))],
        out_specs=pl.BlockSpec((1, 1, 1), lambda j: (0, 0, 0),
                               memory_space=pltpu.SMEM),
        out_shape=jax.ShapeDtypeStruct((1, 1, 1), jnp.float32),
        scratch_shapes=[pltpu.SMEM((1, 1), jnp.float32)],
    )(xt)

    out = pl.pallas_call(
        functools.partial(_tc_fin, n=b * c),
        out_specs=pl.BlockSpec((1, 1, 1), lambda: (0, 0, 0),
                               memory_space=pltpu.SMEM),
        out_shape=jax.ShapeDtypeStruct((1, 1, 1), jnp.float32),
    )(part, tiles, lbl.reshape(b, 1))
    return out.reshape(())
